# baseline (speedup 1.0000x reference)
import functools

import jax
import jax.numpy as jnp
from jax import lax
from jax.experimental import pallas as pl
from jax.experimental.pallas import tpu as pltpu
from jax.experimental.pallas import tpu_sc as plsc

NC = 2
NS = 16
NW = NC * NS
CH = 128
D = 128
SEG_NBUF = 2
GAT_NBUF = 3
RING = SEG_NBUF * GAT_NBUF


def _mesh():
    return plsc.VectorSubcoreMesh(core_axis_name="c", subcore_axis_name="s")


def _make_deg_kernel(npad, nchunks, rows_per_tile):

    @functools.partial(
        pl.kernel,
        mesh=_mesh(),
        out_type=jax.ShapeDtypeStruct((NC * npad,), jnp.float32),
        scratch_types=[
            pltpu.VMEM((1, CH), jnp.int32),
            pltpu.VMEM((CH,), jnp.float32),
            pltpu.VMEM((rows_per_tile,), jnp.float32),
            pltpu.VMEM_SHARED((npad,), jnp.float32),
        ],
    )
    def deg_kernel(dst_hbm, out_hbm, idx_d, ones_v, zbuf, acc_sh):
        c = lax.axis_index("c")
        s = lax.axis_index("s")
        w = s * NC + c

        def zwrite(i, carry):
            zbuf[pl.ds(16 * i, 16)] = jnp.zeros((16,), jnp.float32)
            return carry

        lax.fori_loop(0, rows_per_tile // 16, zwrite, 0)

        def owrite(i, carry):
            ones_v[pl.ds(16 * i, 16)] = jnp.ones((16,), jnp.float32)
            return carry

        lax.fori_loop(0, CH // 16, owrite, 0)
        osl = pl.ds(s * rows_per_tile, rows_per_tile)
        pltpu.sync_copy(zbuf, acc_sh.at[osl])
        plsc.subcore_barrier()

        def body(i, carry):
            pltpu.sync_copy(dst_hbm.at[w, i], idx_d)
            pltpu.sync_copy(ones_v, acc_sh.at[idx_d.at[0]], add=True)
            return carry

        lax.fori_loop(0, nchunks, body, 0)
        plsc.subcore_barrier()
        pltpu.sync_copy(acc_sh.at[osl], zbuf)
        pltpu.sync_copy(zbuf, out_hbm.at[pl.ds(c * npad + s * rows_per_tile,
                                               rows_per_tile)])

    return deg_kernel


def _make_segsum_kernel(npad, nchunks, rows_per_tile):
    nbuf = SEG_NBUF
    nouter = nchunks // nbuf
    nfan = rows_per_tile // CH

    @functools.partial(
        pl.kernel,
        mesh=_mesh(),
        out_type=jax.ShapeDtypeStruct((NC, npad, D), jnp.float32),
        scratch_types=[
            pltpu.VMEM((CH,), jnp.int32),
            pltpu.VMEM((CH,), jnp.int32),
            pltpu.VMEM((CH,), jnp.int32),
            pltpu.VMEM((CH,), jnp.int32),
            pltpu.VMEM((CH, D), jnp.float32),
            pltpu.VMEM((CH, D), jnp.float32),
            pltpu.VMEM_SHARED((npad, D), jnp.float32),
            pltpu.SemaphoreType.DMA,
            pltpu.SemaphoreType.DMA,
        ],
    )
    def segsum_kernel(g_hbm, src_hbm, dst_hbm, out_hbm,
                      idx_s0, idx_d0, idx_s1, idx_d1, rows0, rows1,
                      acc_sh, sem0, sem1):
        c = lax.axis_index("c")
        s = lax.axis_index("s")
        w = s * NC + c

        def zwrite(i, carry):
            def inner(k, carry2):
                rows0[i, pl.ds(16 * k, 16)] = jnp.zeros((16,), jnp.float32)
                return carry2
            return lax.fori_loop(0, D // 16, inner, carry)

        lax.fori_loop(0, CH, zwrite, 0)
        rbase = s * rows_per_tile

        def zfan(j, carry):
            pltpu.sync_copy(rows0, acc_sh.at[pl.ds(rbase + j * CH, CH)])
            return carry

        lax.fori_loop(0, nfan, zfan, 0)
        plsc.subcore_barrier()
        base = w * (nchunks * CH)

        def load_i(i, isv, idv):
            off = base + i * CH
            pltpu.sync_copy(src_hbm.at[pl.ds(off, CH)], isv)
            pltpu.sync_copy(dst_hbm.at[pl.ds(off, CH)], idv)

        def body(i, carry):
            load_i(i, idx_s0, idx_d0)
            pltpu.async_copy(g_hbm.at[idx_s0], rows0, sem0).wait()
            pltpu.sync_copy(rows0, acc_sh.at[idx_d0], add=True)
            return carry

        lax.fori_loop(0, nchunks, body, 0)
        plsc.subcore_barrier()

        def cout(j, carry):
            sl = pl.ds(rbase + j * CH, CH)
            pltpu.sync_copy(acc_sh.at[sl], rows0)
            pltpu.sync_copy(rows0, out_hbm.at[c, sl])
            return carry

        lax.fori_loop(0, nfan, cout, 0)

    return segsum_kernel


def _make_gat_kernel(npad, nchunks, rows_per_tile):
    nbuf = GAT_NBUF
    nouter = nchunks // nbuf

    @functools.partial(
        pl.kernel,
        mesh=_mesh(),
        out_type=jax.ShapeDtypeStruct((NC * 3 * npad,), jnp.float32),
        scratch_types=[
            pltpu.VMEM((nbuf, CH), jnp.int32),
            pltpu.VMEM((nbuf, CH), jnp.int32),
            pltpu.VMEM((nbuf, CH), jnp.float32),
            pltpu.VMEM((nbuf, CH), jnp.float32),
            pltpu.VMEM((nbuf, CH), jnp.float32),
            pltpu.VMEM((nbuf, CH), jnp.float32),
            pltpu.VMEM((nbuf, CH), jnp.float32),
            pltpu.VMEM((nbuf, CH), jnp.float32),
            pltpu.VMEM((nbuf, CH), jnp.float32),
            pltpu.VMEM((rows_per_tile,), jnp.float32),
            pltpu.VMEM((2, 16), jnp.float32),
            pltpu.VMEM_SHARED((npad,), jnp.float32),
            pltpu.VMEM_SHARED((npad,), jnp.float32),
            pltpu.VMEM_SHARED((npad,), jnp.float32),
        ] + [pltpu.SemaphoreType.DMA] * (9 * nbuf),
    )
    def gat_kernel(hl0_hbm, hl1_hbm, hr0_hbm, hr1_hbm, src_hbm, dst_hbm,
                   att_hbm, out_hbm,
                   idx_s, idx_d, a0_v, a1_v, b0_v, b1_v, o0_v, o1_v, o2_v,
                   zbuf, att_v, acc0, acc1, acc2, *sems):
        c = lax.axis_index("c")
        s = lax.axis_index("s")
        w = s * NC + c

        def zwrite(i, carry):
            zbuf[pl.ds(16 * i, 16)] = jnp.zeros((16,), jnp.float32)
            return carry

        lax.fori_loop(0, rows_per_tile // 16, zwrite, 0)
        osl = pl.ds(s * rows_per_tile, rows_per_tile)
        pltpu.sync_copy(zbuf, acc0.at[osl])
        pltpu.sync_copy(zbuf, acc1.at[osl])
        pltpu.sync_copy(zbuf, acc2.at[osl])
        pltpu.sync_copy(att_hbm, att_v)
        plsc.subcore_barrier()
        att0 = att_v[0, :]
        att1 = att_v[1, :]

        def start_i(b, chunk):
            pltpu.async_copy(src_hbm.at[w, chunk], idx_s.at[pl.ds(b, 1)],
                             sems[9 * b + 7])
            pltpu.async_copy(dst_hbm.at[w, chunk], idx_d.at[pl.ds(b, 1)],
                             sems[9 * b + 8])

        def wait_i(b, chunk):
            pltpu.make_async_copy(src_hbm.at[w, chunk], idx_s.at[pl.ds(b, 1)],
                                  sems[9 * b + 7]).wait()
            pltpu.make_async_copy(dst_hbm.at[w, chunk], idx_d.at[pl.ds(b, 1)],
                                  sems[9 * b + 8]).wait()

        def start_g(b):
            isl = idx_s.at[b]
            idl = idx_d.at[b]
            pltpu.async_copy(hl0_hbm.at[idl], a0_v.at[b], sems[9 * b])
            pltpu.async_copy(hl1_hbm.at[idl], a1_v.at[b], sems[9 * b + 1])
            pltpu.async_copy(hr0_hbm.at[isl], b0_v.at[b], sems[9 * b + 2])
            pltpu.async_copy(hr1_hbm.at[isl], b1_v.at[b], sems[9 * b + 3])

        def wait_g(b):
            isl = idx_s.at[b]
            idl = idx_d.at[b]
            pltpu.make_async_copy(hl0_hbm.at[idl], a0_v.at[b],
                                  sems[9 * b]).wait()
            pltpu.make_async_copy(hl1_hbm.at[idl], a1_v.at[b],
                                  sems[9 * b + 1]).wait()
            pltpu.make_async_copy(hr0_hbm.at[isl], b0_v.at[b],
                                  sems[9 * b + 2]).wait()
            pltpu.make_async_copy(hr1_hbm.at[isl], b1_v.at[b],
                                  sems[9 * b + 3]).wait()

        def compute(b):
            for j in range(CH // 16):
                sl = pl.ds(16 * j, 16)
                a0 = a0_v[b, sl]
                a1 = a1_v[b, sl]
                b0 = b0_v[b, sl]
                b1 = b1_v[b, sl]
                z0 = a0 + b0
                z0 = jnp.where(z0 > 0, z0, 0.2 * z0)
                z1 = a1 + b1
                z1 = jnp.where(z1 > 0, z1, 0.2 * z1)
                p = jnp.exp(att0 * z0 + att1 * z1)
                o0_v[b, sl] = p * b0
                o1_v[b, sl] = p * b1
                o2_v[b, sl] = p

        def start_s(b):
            idl = idx_d.at[b]
            pltpu.async_copy(o0_v.at[b], acc0.at[idl], sems[9 * b + 4],
                             add=True)
            pltpu.async_copy(o1_v.at[b], acc1.at[idl], sems[9 * b + 5],
                             add=True)
            pltpu.async_copy(o2_v.at[b], acc2.at[idl], sems[9 * b + 6],
                             add=True)

        def wait_s(b):
            idl = idx_d.at[b]
            pltpu.make_async_copy(o0_v.at[b], acc0.at[idl],
                                  sems[9 * b + 4]).wait()
            pltpu.make_async_copy(o1_v.at[b], acc1.at[idl],
                                  sems[9 * b + 5]).wait()
            pltpu.make_async_copy(o2_v.at[b], acc2.at[idl],
                                  sems[9 * b + 6]).wait()

        for b in range(nbuf):
            start_i(b, b)
        for b in range(nbuf):
            wait_i(b, b)
            start_g(b)

        def outer(o, carry):
            g0 = o * nbuf
            for b in range(nbuf):
                wait_g(b)
                compute(b)
                start_s(b)
            for b in range(nbuf):
                wait_s(b)
                start_i(b, g0 + nbuf + b)
            for b in range(nbuf):
                wait_i(b, g0 + nbuf + b)
                start_g(b)
            return carry

        lax.fori_loop(0, nouter - 1, outer, 0)
        for b in range(nbuf):
            wait_g(b)
            compute(b)
            start_s(b)
        for b in range(nbuf):
            wait_s(b)
        plsc.subcore_barrier()
        ob = c * 3 * npad + s * rows_per_tile
        pltpu.sync_copy(acc0.at[osl], zbuf)
        pltpu.sync_copy(zbuf, out_hbm.at[pl.ds(ob, rows_per_tile)])
        pltpu.sync_copy(acc1.at[osl], zbuf)
        pltpu.sync_copy(zbuf, out_hbm.at[pl.ds(ob + npad, rows_per_tile)])
        pltpu.sync_copy(acc2.at[osl], zbuf)
        pltpu.sync_copy(zbuf, out_hbm.at[pl.ds(ob + 2 * npad, rows_per_tile)])

    return gat_kernel


def _dinv(dega_ref, degb_ref):
    deg = dega_ref[...] + degb_ref[...]
    return jnp.where(deg > 0, lax.rsqrt(deg), 0.0)


def _mm_scale_body(x_ref, w_ref, dega_ref, degb_ref, o_ref):
    o_ref[...] = _dinv(dega_ref, degb_ref) * jnp.dot(
        x_ref[...], w_ref[...], preferred_element_type=jnp.float32)


def _elu_mm_scale_body(sa_ref, sb_ref, dega_ref, degb_ref, w_ref, b_ref, o_ref):
    dinv = _dinv(dega_ref, degb_ref)
    h = dinv * (sa_ref[...] + sb_ref[...]) + b_ref[...]
    h = jnp.where(h > 0, h, jnp.exp(jnp.minimum(h, 0.0)) - 1.0)
    o_ref[...] = dinv * jnp.dot(h, w_ref[...], preferred_element_type=jnp.float32)


def _elu_mm_body(sa_ref, sb_ref, dega_ref, degb_ref, w_ref, b_ref, o_ref):
    dinv = _dinv(dega_ref, degb_ref)
    h = dinv * (sa_ref[...] + sb_ref[...]) + b_ref[...]
    h = jnp.where(h > 0, h, jnp.exp(jnp.minimum(h, 0.0)) - 1.0)
    o_ref[...] = jnp.dot(h, w_ref[...], preferred_element_type=jnp.float32)


def _softmax_norm_body(ga_ref, gb_ref, b_ref, o_ref):
    num = ga_ref[0:2, :] + gb_ref[0:2, :]
    den = ga_ref[2:3, :] + gb_ref[2:3, :]
    o_ref[...] = num / (den + 1e-16) + b_ref[...]


def _tc_call(body, out_shape, *args):
    return pl.pallas_call(body, out_shape=out_shape)(*args)


def kernel(x, edge_index, W0, b0, W1, b1, Wl, Wr, att, b_out):
    n, d = x.shape
    e = edge_index.shape[1]
    el = e + n
    rows_per_tile = -(-(-(-(n + 1) // NS)) // CH) * CH
    npad = NS * rows_per_tile
    per_tile = -(-el // (NW * CH * RING)) * (CH * RING)
    el_pad = NW * per_tile
    nchunks = per_tile // CH

    loops = jnp.arange(n, dtype=jnp.int32)
    src = jnp.concatenate([edge_index[0].astype(jnp.int32), loops])
    dst = jnp.concatenate([edge_index[1].astype(jnp.int32), loops])
    pad_e = el_pad - el
    src = jnp.concatenate([src, jnp.full((pad_e,), n, jnp.int32)])
    dst = jnp.concatenate([dst, jnp.full((pad_e,), n, jnp.int32)])
    src4 = src.reshape(NW, nchunks, 1, CH)
    dst4 = dst.reshape(NW, nchunks, 1, CH)

    xpad = jnp.zeros((npad, d), jnp.float32).at[:n].set(x)
    att_rep = jnp.broadcast_to(att.reshape(2, 1), (2, 16)).astype(jnp.float32)
    w_gat = jnp.zeros((d, 8), jnp.float32).at[:, 0:2].set(Wl).at[:, 2:4].set(Wr)

    deg_k = _make_deg_kernel(npad, nchunks, rows_per_tile)
    seg_k = _make_segsum_kernel(npad, nchunks, rows_per_tile)
    gat_k = _make_gat_kernel(npad, nchunks, rows_per_tile)

    dacc = deg_k(dst4)
    dega = dacc[:npad].reshape(npad, 1)
    degb = dacc[npad:].reshape(npad, 1)

    g0 = _tc_call(_mm_scale_body,
                  jax.ShapeDtypeStruct((npad, D), jnp.float32),
                  xpad, W0, dega, degb)
    s0 = seg_k(g0, src, dst)
    g1 = _tc_call(_elu_mm_scale_body,
                  jax.ShapeDtypeStruct((npad, D), jnp.float32),
                  s0[0], s0[1], dega, degb, W1, b0)
    s1 = seg_k(g1, src, dst)
    t8 = _tc_call(_elu_mm_body,
                  jax.ShapeDtypeStruct((npad, 8), jnp.float32),
                  s1[0], s1[1], dega, degb, w_gat, b1)
    t4 = jnp.transpose(t8[:, :4])
    gacc = gat_k(t4[0], t4[1], t4[2], t4[3], src4, dst4, att_rep)
    ga = gacc[:3 * npad].reshape(3, npad)
    gb = gacc[3 * npad:].reshape(3, npad)
    res = _tc_call(_softmax_norm_body,
                   jax.ShapeDtypeStruct((2, npad), jnp.float32),
                   ga, gb, b_out.reshape(2, 1).astype(jnp.float32))
    return jnp.transpose(res)[:n]


# speedup vs baseline: 1.9508x; 1.9508x over previous
import functools

import jax
import jax.numpy as jnp
from jax import lax
from jax.experimental import pallas as pl
from jax.experimental.pallas import tpu as pltpu
from jax.experimental.pallas import tpu_sc as plsc

NC = 2
NS = 16
NW = NC * NS
CH = 128
D = 128


def _mesh():
    return plsc.VectorSubcoreMesh(core_axis_name="c", subcore_axis_name="s")


def _make_deg_kernel(npad, el_pad, per_tile, rows_per_tile):
    nchunks = per_tile // CH

    @functools.partial(
        pl.kernel,
        mesh=_mesh(),
        out_type=jax.ShapeDtypeStruct((NC * npad,), jnp.float32),
        scratch_types=[
            pltpu.VMEM((CH,), jnp.int32),
            pltpu.VMEM((CH,), jnp.float32),
            pltpu.VMEM((rows_per_tile,), jnp.float32),
            pltpu.VMEM_SHARED((npad,), jnp.float32),
        ],
    )
    def deg_kernel(dst_hbm, out_hbm, idx_d, ones_v, zbuf, acc_sh):
        c = lax.axis_index("c")
        s = lax.axis_index("s")
        w = s * NC + c

        def zwrite(i, carry):
            zbuf[pl.ds(16 * i, 16)] = jnp.zeros((16,), jnp.float32)
            return carry

        lax.fori_loop(0, rows_per_tile // 16, zwrite, 0)

        def owrite(i, carry):
            ones_v[pl.ds(16 * i, 16)] = jnp.ones((16,), jnp.float32)
            return carry

        lax.fori_loop(0, CH // 16, owrite, 0)
        osl = pl.ds(s * rows_per_tile, rows_per_tile)
        pltpu.sync_copy(zbuf, acc_sh.at[osl])
        plsc.subcore_barrier()
        base = w * per_tile

        def body(i, carry):
            off = base + i * CH
            pltpu.sync_copy(dst_hbm.at[pl.ds(off, CH)], idx_d)
            pltpu.sync_copy(ones_v, acc_sh.at[idx_d], add=True)
            return carry

        lax.fori_loop(0, nchunks, body, 0)
        plsc.subcore_barrier()
        pltpu.sync_copy(acc_sh.at[osl], zbuf)
        pltpu.sync_copy(zbuf, out_hbm.at[pl.ds(c * npad + s * rows_per_tile,
                                               rows_per_tile)])

    return deg_kernel


def _make_segsum_kernel(npad, el_pad, per_tile, rows_per_tile):
    nchunks = per_tile // CH
    nfan = rows_per_tile // CH

    @functools.partial(
        pl.kernel,
        mesh=_mesh(),
        out_type=jax.ShapeDtypeStruct((NC, npad, D), jnp.float32),
        scratch_types=[
            pltpu.VMEM((CH,), jnp.int32),
            pltpu.VMEM((CH,), jnp.int32),
            pltpu.VMEM((CH, D), jnp.float32),
            pltpu.VMEM_SHARED((npad, D), jnp.float32),
            pltpu.SemaphoreType.DMA,
        ],
    )
    def segsum_kernel(g_hbm, src_hbm, dst_hbm, out_hbm,
                      idx_s, idx_d, rows_v, acc_sh, sem):
        c = lax.axis_index("c")
        s = lax.axis_index("s")
        w = s * NC + c

        def zwrite(i, carry):
            def inner(k, carry2):
                rows_v[i, pl.ds(16 * k, 16)] = jnp.zeros((16,), jnp.float32)
                return carry2
            return lax.fori_loop(0, D // 16, inner, carry)

        lax.fori_loop(0, CH, zwrite, 0)
        rbase = s * rows_per_tile

        def zfan(j, carry):
            pltpu.sync_copy(rows_v, acc_sh.at[pl.ds(rbase + j * CH, CH)])
            return carry

        lax.fori_loop(0, nfan, zfan, 0)
        plsc.subcore_barrier()
        base = w * per_tile

        def body(i, carry):
            off = base + i * CH
            pltpu.sync_copy(src_hbm.at[pl.ds(off, CH)], idx_s)
            pltpu.sync_copy(dst_hbm.at[pl.ds(off, CH)], idx_d)
            pltpu.async_copy(g_hbm.at[idx_s], rows_v, sem).wait()
            pltpu.sync_copy(rows_v, acc_sh.at[idx_d], add=True)
            return carry

        lax.fori_loop(0, nchunks, body, 0)
        plsc.subcore_barrier()

        def cout(j, carry):
            sl = pl.ds(rbase + j * CH, CH)
            pltpu.sync_copy(acc_sh.at[sl], rows_v)
            pltpu.sync_copy(rows_v, out_hbm.at[c, sl])
            return carry

        lax.fori_loop(0, nfan, cout, 0)

    return segsum_kernel


def _make_gat_kernel(npad, el_pad, per_tile, rows_per_tile):
    nchunks = per_tile // CH

    @functools.partial(
        pl.kernel,
        mesh=_mesh(),
        out_type=jax.ShapeDtypeStruct((NC * 3 * npad,), jnp.float32),
        scratch_types=[
            pltpu.VMEM((CH,), jnp.int32),
            pltpu.VMEM((CH,), jnp.int32),
            pltpu.VMEM((CH,), jnp.float32),
            pltpu.VMEM((CH,), jnp.float32),
            pltpu.VMEM((CH,), jnp.float32),
            pltpu.VMEM((CH,), jnp.float32),
            pltpu.VMEM((CH,), jnp.float32),
            pltpu.VMEM((CH,), jnp.float32),
            pltpu.VMEM((CH,), jnp.float32),
            pltpu.VMEM((rows_per_tile,), jnp.float32),
            pltpu.VMEM((2, 16), jnp.float32),
            pltpu.VMEM_SHARED((npad,), jnp.float32),
            pltpu.VMEM_SHARED((npad,), jnp.float32),
            pltpu.VMEM_SHARED((npad,), jnp.float32),
            pltpu.SemaphoreType.DMA,
            pltpu.SemaphoreType.DMA,
            pltpu.SemaphoreType.DMA,
            pltpu.SemaphoreType.DMA,
        ],
    )
    def gat_kernel(hl0_hbm, hl1_hbm, hr0_hbm, hr1_hbm, src_hbm, dst_hbm,
                   att_hbm, out_hbm,
                   idx_s, idx_d, a0_v, a1_v, b0_v, b1_v, o0_v, o1_v, o2_v,
                   zbuf, att_v, acc0, acc1, acc2, sem0, sem1, sem2, sem3):
        c = lax.axis_index("c")
        s = lax.axis_index("s")
        w = s * NC + c

        def zwrite(i, carry):
            zbuf[pl.ds(16 * i, 16)] = jnp.zeros((16,), jnp.float32)
            return carry

        lax.fori_loop(0, rows_per_tile // 16, zwrite, 0)
        osl = pl.ds(s * rows_per_tile, rows_per_tile)
        pltpu.sync_copy(zbuf, acc0.at[osl])
        pltpu.sync_copy(zbuf, acc1.at[osl])
        pltpu.sync_copy(zbuf, acc2.at[osl])
        pltpu.sync_copy(att_hbm, att_v)
        plsc.subcore_barrier()
        att0 = att_v[0, :]
        att1 = att_v[1, :]
        base = w * per_tile

        def body(i, carry):
            off = base + i * CH
            pltpu.sync_copy(src_hbm.at[pl.ds(off, CH)], idx_s)
            pltpu.sync_copy(dst_hbm.at[pl.ds(off, CH)], idx_d)
            cp0 = pltpu.async_copy(hl0_hbm.at[idx_d], a0_v, sem0)
            cp1 = pltpu.async_copy(hl1_hbm.at[idx_d], a1_v, sem1)
            cp2 = pltpu.async_copy(hr0_hbm.at[idx_s], b0_v, sem2)
            cp3 = pltpu.async_copy(hr1_hbm.at[idx_s], b1_v, sem3)
            cp0.wait()
            cp1.wait()
            cp2.wait()
            cp3.wait()
            for j in range(CH // 16):
                sl = pl.ds(16 * j, 16)
                a0 = a0_v[sl]
                a1 = a1_v[sl]
                b0 = b0_v[sl]
                b1 = b1_v[sl]
                z0 = a0 + b0
                z0 = jnp.where(z0 > 0, z0, 0.2 * z0)
                z1 = a1 + b1
                z1 = jnp.where(z1 > 0, z1, 0.2 * z1)
                p = jnp.exp(att0 * z0 + att1 * z1)
                o0_v[sl] = p * b0
                o1_v[sl] = p * b1
                o2_v[sl] = p
            pltpu.sync_copy(o0_v, acc0.at[idx_d], add=True)
            pltpu.sync_copy(o1_v, acc1.at[idx_d], add=True)
            pltpu.sync_copy(o2_v, acc2.at[idx_d], add=True)
            return carry

        lax.fori_loop(0, nchunks, body, 0)
        plsc.subcore_barrier()
        ob = c * 3 * npad + s * rows_per_tile
        pltpu.sync_copy(acc0.at[osl], zbuf)
        pltpu.sync_copy(zbuf, out_hbm.at[pl.ds(ob, rows_per_tile)])
        pltpu.sync_copy(acc1.at[osl], zbuf)
        pltpu.sync_copy(zbuf, out_hbm.at[pl.ds(ob + npad, rows_per_tile)])
        pltpu.sync_copy(acc2.at[osl], zbuf)
        pltpu.sync_copy(zbuf, out_hbm.at[pl.ds(ob + 2 * npad, rows_per_tile)])

    return gat_kernel


def _dinv(dega_ref, degb_ref):
    deg = dega_ref[...] + degb_ref[...]
    return jnp.where(deg > 0, lax.rsqrt(deg), 0.0)


def _mm_scale_body(x_ref, w_ref, dega_ref, degb_ref, o_ref):
    o_ref[...] = _dinv(dega_ref, degb_ref) * jnp.dot(
        x_ref[...], w_ref[...], preferred_element_type=jnp.float32)


def _elu_mm_scale_body(sa_ref, sb_ref, dega_ref, degb_ref, w_ref, b_ref, o_ref):
    dinv = _dinv(dega_ref, degb_ref)
    h = dinv * (sa_ref[...] + sb_ref[...]) + b_ref[...]
    h = jnp.where(h > 0, h, jnp.exp(jnp.minimum(h, 0.0)) - 1.0)
    o_ref[...] = dinv * jnp.dot(h, w_ref[...], preferred_element_type=jnp.float32)


def _elu_mm_body(sa_ref, sb_ref, dega_ref, degb_ref, w_ref, b_ref, o_ref):
    dinv = _dinv(dega_ref, degb_ref)
    h = dinv * (sa_ref[...] + sb_ref[...]) + b_ref[...]
    h = jnp.where(h > 0, h, jnp.exp(jnp.minimum(h, 0.0)) - 1.0)
    o_ref[...] = jnp.dot(h, w_ref[...], preferred_element_type=jnp.float32)


def _softmax_norm_body(ga_ref, gb_ref, b_ref, o_ref):
    num = ga_ref[0:2, :] + gb_ref[0:2, :]
    den = ga_ref[2:3, :] + gb_ref[2:3, :]
    o_ref[...] = num / (den + 1e-16) + b_ref[...]


def _tc_call(body, out_shape, *args):
    return pl.pallas_call(body, out_shape=out_shape)(*args)


def kernel(x, edge_index, W0, b0, W1, b1, Wl, Wr, att, b_out):
    n, d = x.shape
    e = edge_index.shape[1]
    el = e + n
    rows_per_tile = -(-(-(-(n + 1) // NS)) // CH) * CH
    npad = NS * rows_per_tile
    per_tile = -(-el // (NW * CH)) * CH
    el_pad = NW * per_tile

    loops = jnp.arange(n, dtype=jnp.int32)
    src = jnp.concatenate([edge_index[0].astype(jnp.int32), loops])
    dst = jnp.concatenate([edge_index[1].astype(jnp.int32), loops])
    pad_e = el_pad - el
    src = jnp.concatenate([src, jnp.full((pad_e,), n, jnp.int32)])
    dst = jnp.concatenate([dst, jnp.full((pad_e,), n, jnp.int32)])

    xpad = jnp.zeros((npad, d), jnp.float32).at[:n].set(x)
    att_rep = jnp.broadcast_to(att.reshape(2, 1), (2, 16)).astype(jnp.float32)
    w_gat = jnp.zeros((d, 8), jnp.float32).at[:, 0:2].set(Wl).at[:, 2:4].set(Wr)

    deg_k = _make_deg_kernel(npad, el_pad, per_tile, rows_per_tile)
    seg_k = _make_segsum_kernel(npad, el_pad, per_tile, rows_per_tile)
    gat_k = _make_gat_kernel(npad, el_pad, per_tile, rows_per_tile)

    dacc = deg_k(dst)
    dega = dacc[:npad].reshape(npad, 1)
    degb = dacc[npad:].reshape(npad, 1)

    g0 = _tc_call(_mm_scale_body,
                  jax.ShapeDtypeStruct((npad, D), jnp.float32),
                  xpad, W0, dega, degb)
    s0 = seg_k(g0, src, dst)
    g1 = _tc_call(_elu_mm_scale_body,
                  jax.ShapeDtypeStruct((npad, D), jnp.float32),
                  s0[0], s0[1], dega, degb, W1, b0)
    s1 = seg_k(g1, src, dst)
    t8 = _tc_call(_elu_mm_body,
                  jax.ShapeDtypeStruct((npad, 8), jnp.float32),
                  s1[0], s1[1], dega, degb, w_gat, b1)
    t4 = jnp.transpose(t8[:, :4])
    gacc = gat_k(t4[0], t4[1], t4[2], t4[3], src, dst, att_rep)
    ga = gacc[:3 * npad].reshape(3, npad)
    gb = gacc[3 * npad:].reshape(3, npad)
    res = _tc_call(_softmax_norm_body,
                   jax.ShapeDtypeStruct((2, npad), jnp.float32),
                   ga, gb, b_out.reshape(2, 1).astype(jnp.float32))
    return jnp.transpose(res)[:n]


# speedup vs baseline: 2.1788x; 1.1169x over previous
import functools

import jax
import jax.numpy as jnp
from jax import lax
from jax.experimental import pallas as pl
from jax.experimental.pallas import tpu as pltpu
from jax.experimental.pallas import tpu_sc as plsc

NC = 2
NS = 16
NW = NC * NS
CH = 128
D = 128


def _mesh():
    return plsc.VectorSubcoreMesh(core_axis_name="c", subcore_axis_name="s")


def _make_deg_kernel(npad, el_pad, per_tile, rows_per_tile):
    nchunks = per_tile // CH

    @functools.partial(
        pl.kernel,
        mesh=_mesh(),
        out_type=jax.ShapeDtypeStruct((NC * npad,), jnp.float32),
        scratch_types=[
            pltpu.VMEM((CH,), jnp.int32),
            pltpu.VMEM((CH,), jnp.float32),
            pltpu.VMEM((rows_per_tile,), jnp.float32),
            pltpu.VMEM_SHARED((npad,), jnp.float32),
        ],
    )
    def deg_kernel(dst_hbm, out_hbm, idx_d, ones_v, zbuf, acc_sh):
        c = lax.axis_index("c")
        s = lax.axis_index("s")
        w = s * NC + c

        def zwrite(i, carry):
            zbuf[pl.ds(16 * i, 16)] = jnp.zeros((16,), jnp.float32)
            return carry

        lax.fori_loop(0, rows_per_tile // 16, zwrite, 0)

        def owrite(i, carry):
            ones_v[pl.ds(16 * i, 16)] = jnp.ones((16,), jnp.float32)
            return carry

        lax.fori_loop(0, CH // 16, owrite, 0)
        osl = pl.ds(s * rows_per_tile, rows_per_tile)
        pltpu.sync_copy(zbuf, acc_sh.at[osl])
        plsc.subcore_barrier()
        base = w * per_tile

        def body(i, carry):
            off = base + i * CH
            pltpu.sync_copy(dst_hbm.at[pl.ds(off, CH)], idx_d)
            pltpu.sync_copy(ones_v, acc_sh.at[idx_d], add=True)
            return carry

        lax.fori_loop(0, nchunks, body, 0)
        plsc.subcore_barrier()
        pltpu.sync_copy(acc_sh.at[osl], zbuf)
        pltpu.sync_copy(zbuf, out_hbm.at[pl.ds(c * npad + s * rows_per_tile,
                                               rows_per_tile)])

    return deg_kernel


def _make_segsum_kernel(npad, el_pad, per_tile, rows_per_tile):
    nchunks = per_tile // CH
    nfan = rows_per_tile // CH

    @functools.partial(
        pl.kernel,
        mesh=_mesh(),
        out_type=jax.ShapeDtypeStruct((NC, npad, D), jnp.float32),
        scratch_types=[
            pltpu.VMEM((CH,), jnp.int32),
            pltpu.VMEM((CH,), jnp.int32),
            pltpu.VMEM((CH, D), jnp.float32),
            pltpu.VMEM_SHARED((npad, D), jnp.float32),
            pltpu.SemaphoreType.DMA,
        ],
    )
    def segsum_kernel(g_hbm, src_hbm, dst_hbm, out_hbm,
                      idx_s, idx_d, rows_v, acc_sh, sem):
        c = lax.axis_index("c")
        s = lax.axis_index("s")
        w = s * NC + c

        def zwrite(i, carry):
            def inner(k, carry2):
                rows_v[i, pl.ds(16 * k, 16)] = jnp.zeros((16,), jnp.float32)
                return carry2
            return lax.fori_loop(0, D // 16, inner, carry)

        lax.fori_loop(0, CH, zwrite, 0)
        rbase = s * rows_per_tile

        def zfan(j, carry):
            pltpu.sync_copy(rows_v, acc_sh.at[pl.ds(rbase + j * CH, CH)])
            return carry

        lax.fori_loop(0, nfan, zfan, 0)
        plsc.subcore_barrier()
        base = w * per_tile

        def body(i, carry):
            off = base + i * CH
            pltpu.sync_copy(src_hbm.at[pl.ds(off, CH)], idx_s)
            pltpu.sync_copy(dst_hbm.at[pl.ds(off, CH)], idx_d)
            pltpu.async_copy(g_hbm.at[idx_s], rows_v, sem).wait()
            pltpu.sync_copy(rows_v, acc_sh.at[idx_d], add=True)
            return carry

        lax.fori_loop(0, nchunks, body, 0)
        plsc.subcore_barrier()

        def cout(j, carry):
            sl = pl.ds(rbase + j * CH, CH)
            pltpu.sync_copy(acc_sh.at[sl], rows_v)
            pltpu.sync_copy(rows_v, out_hbm.at[c, sl])
            return carry

        lax.fori_loop(0, nfan, cout, 0)

    return segsum_kernel


def _make_gat_kernel(npad, el_pad, per_tile, rows_per_tile):
    nchunks = per_tile // CH
    nbuf = 3
    nouter = nchunks // nbuf

    @functools.partial(
        pl.kernel,
        mesh=_mesh(),
        out_type=jax.ShapeDtypeStruct((NC * 3 * npad,), jnp.float32),
        scratch_types=[
            pltpu.VMEM((nbuf, CH), jnp.int32),
            pltpu.VMEM((nbuf, CH), jnp.int32),
            pltpu.VMEM((nbuf, CH), jnp.float32),
            pltpu.VMEM((nbuf, CH), jnp.float32),
            pltpu.VMEM((nbuf, CH), jnp.float32),
            pltpu.VMEM((nbuf, CH), jnp.float32),
            pltpu.VMEM((nbuf, CH), jnp.float32),
            pltpu.VMEM((nbuf, CH), jnp.float32),
            pltpu.VMEM((nbuf, CH), jnp.float32),
            pltpu.VMEM((rows_per_tile,), jnp.float32),
            pltpu.VMEM((2, 16), jnp.float32),
            pltpu.VMEM_SHARED((npad,), jnp.float32),
            pltpu.VMEM_SHARED((npad,), jnp.float32),
            pltpu.VMEM_SHARED((npad,), jnp.float32),
        ] + [pltpu.SemaphoreType.DMA] * 27,
    )
    def gat_kernel(hl0_hbm, hl1_hbm, hr0_hbm, hr1_hbm, src_hbm, dst_hbm,
                   att_hbm, out_hbm,
                   idx_s, idx_d, a0_v, a1_v, b0_v, b1_v, o0_v, o1_v, o2_v,
                   zbuf, att_v, acc0, acc1, acc2, *sems):
        c = lax.axis_index("c")
        s = lax.axis_index("s")
        w = s * NC + c

        def zwrite(i, carry):
            zbuf[pl.ds(16 * i, 16)] = jnp.zeros((16,), jnp.float32)
            return carry

        lax.fori_loop(0, rows_per_tile // 16, zwrite, 0)
        osl = pl.ds(s * rows_per_tile, rows_per_tile)
        pltpu.sync_copy(zbuf, acc0.at[osl])
        pltpu.sync_copy(zbuf, acc1.at[osl])
        pltpu.sync_copy(zbuf, acc2.at[osl])
        pltpu.sync_copy(att_hbm, att_v)
        plsc.subcore_barrier()
        att0 = att_v[0, :]
        att1 = att_v[1, :]
        base = w * per_tile

        def start_i(b, chunk):
            off = base + chunk * CH
            pltpu.async_copy(src_hbm.at[pl.ds(off, CH)],
                             idx_s.at[b], sems[9 * b + 7])
            pltpu.async_copy(dst_hbm.at[pl.ds(off, CH)],
                             idx_d.at[b], sems[9 * b + 8])

        def wait_i(b, chunk):
            off = base + chunk * CH
            pltpu.make_async_copy(src_hbm.at[pl.ds(off, CH)],
                                  idx_s.at[b], sems[9 * b + 7]).wait()
            pltpu.make_async_copy(dst_hbm.at[pl.ds(off, CH)],
                                  idx_d.at[b], sems[9 * b + 8]).wait()

        def start_g(b):
            isl = idx_s.at[b]
            idl = idx_d.at[b]
            pltpu.async_copy(hl0_hbm.at[idl], a0_v.at[b], sems[9 * b])
            pltpu.async_copy(hl1_hbm.at[idl], a1_v.at[b], sems[9 * b + 1])
            pltpu.async_copy(hr0_hbm.at[isl], b0_v.at[b], sems[9 * b + 2])
            pltpu.async_copy(hr1_hbm.at[isl], b1_v.at[b], sems[9 * b + 3])

        def wait_g(b):
            isl = idx_s.at[b]
            idl = idx_d.at[b]
            pltpu.make_async_copy(hl0_hbm.at[idl], a0_v.at[b],
                                  sems[9 * b]).wait()
            pltpu.make_async_copy(hl1_hbm.at[idl], a1_v.at[b],
                                  sems[9 * b + 1]).wait()
            pltpu.make_async_copy(hr0_hbm.at[isl], b0_v.at[b],
                                  sems[9 * b + 2]).wait()
            pltpu.make_async_copy(hr1_hbm.at[isl], b1_v.at[b],
                                  sems[9 * b + 3]).wait()

        def compute(b):
            for j in range(CH // 16):
                sl = pl.ds(16 * j, 16)
                a0 = a0_v[b, sl]
                a1 = a1_v[b, sl]
                b0 = b0_v[b, sl]
                b1 = b1_v[b, sl]
                z0 = a0 + b0
                z0 = jnp.where(z0 > 0, z0, 0.2 * z0)
                z1 = a1 + b1
                z1 = jnp.where(z1 > 0, z1, 0.2 * z1)
                p = jnp.exp(att0 * z0 + att1 * z1)
                o0_v[b, sl] = p * b0
                o1_v[b, sl] = p * b1
                o2_v[b, sl] = p

        def start_s(b):
            idl = idx_d.at[b]
            pltpu.async_copy(o0_v.at[b], acc0.at[idl], sems[9 * b + 4],
                             add=True)
            pltpu.async_copy(o1_v.at[b], acc1.at[idl], sems[9 * b + 5],
                             add=True)
            pltpu.async_copy(o2_v.at[b], acc2.at[idl], sems[9 * b + 6],
                             add=True)

        def wait_s(b):
            idl = idx_d.at[b]
            pltpu.make_async_copy(o0_v.at[b], acc0.at[idl],
                                  sems[9 * b + 4]).wait()
            pltpu.make_async_copy(o1_v.at[b], acc1.at[idl],
                                  sems[9 * b + 5]).wait()
            pltpu.make_async_copy(o2_v.at[b], acc2.at[idl],
                                  sems[9 * b + 6]).wait()

        for b in range(nbuf):
            start_i(b, b)
        for b in range(nbuf):
            wait_i(b, b)
            start_g(b)

        def outer(o, carry):
            g0 = o * nbuf
            for b in range(nbuf):
                wait_g(b)
                compute(b)
                start_s(b)
            for b in range(nbuf):
                wait_s(b)
                start_i(b, g0 + nbuf + b)
            for b in range(nbuf):
                wait_i(b, g0 + nbuf + b)
                start_g(b)
            return carry

        lax.fori_loop(0, nouter - 1, outer, 0)
        for b in range(nbuf):
            wait_g(b)
            compute(b)
            start_s(b)
        for b in range(nbuf):
            wait_s(b)
        plsc.subcore_barrier()
        ob = c * 3 * npad + s * rows_per_tile
        pltpu.sync_copy(acc0.at[osl], zbuf)
        pltpu.sync_copy(zbuf, out_hbm.at[pl.ds(ob, rows_per_tile)])
        pltpu.sync_copy(acc1.at[osl], zbuf)
        pltpu.sync_copy(zbuf, out_hbm.at[pl.ds(ob + npad, rows_per_tile)])
        pltpu.sync_copy(acc2.at[osl], zbuf)
        pltpu.sync_copy(zbuf, out_hbm.at[pl.ds(ob + 2 * npad, rows_per_tile)])

    return gat_kernel


def _dinv(dega_ref, degb_ref):
    deg = dega_ref[...] + degb_ref[...]
    return jnp.where(deg > 0, lax.rsqrt(deg), 0.0)


def _mm_scale_body(x_ref, w_ref, dega_ref, degb_ref, o_ref):
    o_ref[...] = _dinv(dega_ref, degb_ref) * jnp.dot(
        x_ref[...], w_ref[...], preferred_element_type=jnp.float32)


def _elu_mm_scale_body(sa_ref, sb_ref, dega_ref, degb_ref, w_ref, b_ref, o_ref):
    dinv = _dinv(dega_ref, degb_ref)
    h = dinv * (sa_ref[...] + sb_ref[...]) + b_ref[...]
    h = jnp.where(h > 0, h, jnp.exp(jnp.minimum(h, 0.0)) - 1.0)
    o_ref[...] = dinv * jnp.dot(h, w_ref[...], preferred_element_type=jnp.float32)


def _elu_mm_body(sa_ref, sb_ref, dega_ref, degb_ref, w_ref, b_ref, o_ref):
    dinv = _dinv(dega_ref, degb_ref)
    h = dinv * (sa_ref[...] + sb_ref[...]) + b_ref[...]
    h = jnp.where(h > 0, h, jnp.exp(jnp.minimum(h, 0.0)) - 1.0)
    o_ref[...] = jnp.dot(h, w_ref[...], preferred_element_type=jnp.float32)


def _softmax_norm_body(ga_ref, gb_ref, b_ref, o_ref):
    num = ga_ref[0:2, :] + gb_ref[0:2, :]
    den = ga_ref[2:3, :] + gb_ref[2:3, :]
    o_ref[...] = num / (den + 1e-16) + b_ref[...]


def _tc_call(body, out_shape, *args):
    return pl.pallas_call(body, out_shape=out_shape)(*args)


def kernel(x, edge_index, W0, b0, W1, b1, Wl, Wr, att, b_out):
    n, d = x.shape
    e = edge_index.shape[1]
    el = e + n
    rows_per_tile = -(-(-(-(n + 1) // NS)) // CH) * CH
    npad = NS * rows_per_tile
    per_tile = -(-el // (NW * CH)) * CH
    el_pad = NW * per_tile

    loops = jnp.arange(n, dtype=jnp.int32)
    src = jnp.concatenate([edge_index[0].astype(jnp.int32), loops])
    dst = jnp.concatenate([edge_index[1].astype(jnp.int32), loops])
    pad_e = el_pad - el
    src = jnp.concatenate([src, jnp.full((pad_e,), n, jnp.int32)])
    dst = jnp.concatenate([dst, jnp.full((pad_e,), n, jnp.int32)])

    xpad = jnp.zeros((npad, d), jnp.float32).at[:n].set(x)
    att_rep = jnp.broadcast_to(att.reshape(2, 1), (2, 16)).astype(jnp.float32)
    w_gat = jnp.zeros((d, 8), jnp.float32).at[:, 0:2].set(Wl).at[:, 2:4].set(Wr)

    deg_k = _make_deg_kernel(npad, el_pad, per_tile, rows_per_tile)
    seg_k = _make_segsum_kernel(npad, el_pad, per_tile, rows_per_tile)
    gat_k = _make_gat_kernel(npad, el_pad, per_tile, rows_per_tile)

    dacc = deg_k(dst)
    dega = dacc[:npad].reshape(npad, 1)
    degb = dacc[npad:].reshape(npad, 1)

    g0 = _tc_call(_mm_scale_body,
                  jax.ShapeDtypeStruct((npad, D), jnp.float32),
                  xpad, W0, dega, degb)
    s0 = seg_k(g0, src, dst)
    g1 = _tc_call(_elu_mm_scale_body,
                  jax.ShapeDtypeStruct((npad, D), jnp.float32),
                  s0[0], s0[1], dega, degb, W1, b0)
    s1 = seg_k(g1, src, dst)
    t8 = _tc_call(_elu_mm_body,
                  jax.ShapeDtypeStruct((npad, 8), jnp.float32),
                  s1[0], s1[1], dega, degb, w_gat, b1)
    t4 = jnp.transpose(t8[:, :4])
    gacc = gat_k(t4[0], t4[1], t4[2], t4[3], src, dst, att_rep)
    ga = gacc[:3 * npad].reshape(3, npad)
    gb = gacc[3 * npad:].reshape(3, npad)
    res = _tc_call(_softmax_norm_body,
                   jax.ShapeDtypeStruct((2, npad), jnp.float32),
                   ga, gb, b_out.reshape(2, 1).astype(jnp.float32))
    return jnp.transpose(res)[:n]


# speedup vs baseline: 2.7628x; 1.2680x over previous
import functools

import jax
import jax.numpy as jnp
from jax import lax
from jax.experimental import pallas as pl
from jax.experimental.pallas import tpu as pltpu
from jax.experimental.pallas import tpu_sc as plsc

NC = 2
NS = 16
NW = NC * NS
CH = 128
D = 128


def _mesh():
    return plsc.VectorSubcoreMesh(core_axis_name="c", subcore_axis_name="s")


def _make_deg_kernel(npad, el_pad, per_tile, rows_per_tile):
    nchunks = per_tile // CH

    @functools.partial(
        pl.kernel,
        mesh=_mesh(),
        out_type=jax.ShapeDtypeStruct((NC * npad,), jnp.float32),
        scratch_types=[
            pltpu.VMEM((CH,), jnp.int32),
            pltpu.VMEM((CH,), jnp.float32),
            pltpu.VMEM((rows_per_tile,), jnp.float32),
            pltpu.VMEM_SHARED((npad,), jnp.float32),
        ],
    )
    def deg_kernel(dst_hbm, out_hbm, idx_d, ones_v, zbuf, acc_sh):
        c = lax.axis_index("c")
        s = lax.axis_index("s")
        w = s * NC + c

        def zwrite(i, carry):
            zbuf[pl.ds(16 * i, 16)] = jnp.zeros((16,), jnp.float32)
            return carry

        lax.fori_loop(0, rows_per_tile // 16, zwrite, 0)

        def owrite(i, carry):
            ones_v[pl.ds(16 * i, 16)] = jnp.ones((16,), jnp.float32)
            return carry

        lax.fori_loop(0, CH // 16, owrite, 0)
        osl = pl.ds(s * rows_per_tile, rows_per_tile)
        pltpu.sync_copy(zbuf, acc_sh.at[osl])
        plsc.subcore_barrier()
        base = w * per_tile

        def body(i, carry):
            off = base + i * CH
            pltpu.sync_copy(dst_hbm.at[pl.ds(off, CH)], idx_d)
            pltpu.sync_copy(ones_v, acc_sh.at[idx_d], add=True)
            return carry

        lax.fori_loop(0, nchunks, body, 0)
        plsc.subcore_barrier()
        pltpu.sync_copy(acc_sh.at[osl], zbuf)
        pltpu.sync_copy(zbuf, out_hbm.at[pl.ds(c * npad + s * rows_per_tile,
                                               rows_per_tile)])

    return deg_kernel


def _make_segsum_kernel(npad, el_pad, per_tile, rows_per_tile):
    nchunks = per_tile // CH
    nfan = rows_per_tile // CH

    npair = (nchunks - 1) // 2

    @functools.partial(
        pl.kernel,
        mesh=_mesh(),
        out_type=jax.ShapeDtypeStruct((NC, npad, D), jnp.float32),
        scratch_types=[
            pltpu.VMEM((CH,), jnp.int32),
            pltpu.VMEM((CH,), jnp.int32),
            pltpu.VMEM((CH,), jnp.int32),
            pltpu.VMEM((CH,), jnp.int32),
            pltpu.VMEM((CH, D), jnp.float32),
            pltpu.VMEM((CH, D), jnp.float32),
            pltpu.VMEM_SHARED((npad, D), jnp.float32),
            pltpu.SemaphoreType.DMA,
            pltpu.SemaphoreType.DMA,
        ],
    )
    def segsum_kernel(g_hbm, src_hbm, dst_hbm, out_hbm,
                      idx_s0, idx_d0, idx_s1, idx_d1, rows0, rows1,
                      acc_sh, sem0, sem1):
        c = lax.axis_index("c")
        s = lax.axis_index("s")
        w = s * NC + c

        def zwrite(i, carry):
            def inner(k, carry2):
                rows0[i, pl.ds(16 * k, 16)] = jnp.zeros((16,), jnp.float32)
                return carry2
            return lax.fori_loop(0, D // 16, inner, carry)

        lax.fori_loop(0, CH, zwrite, 0)
        rbase = s * rows_per_tile

        def zfan(j, carry):
            pltpu.sync_copy(rows0, acc_sh.at[pl.ds(rbase + j * CH, CH)])
            return carry

        lax.fori_loop(0, nfan, zfan, 0)
        plsc.subcore_barrier()
        base = w * per_tile

        def load_i(i, isv, idv):
            off = base + i * CH
            pltpu.sync_copy(src_hbm.at[pl.ds(off, CH)], isv)
            pltpu.sync_copy(dst_hbm.at[pl.ds(off, CH)], idv)

        load_i(0, idx_s0, idx_d0)
        pltpu.async_copy(g_hbm.at[idx_s0], rows0, sem0)

        def body(k, carry):
            load_i(2 * k + 1, idx_s1, idx_d1)
            pltpu.async_copy(g_hbm.at[idx_s1], rows1, sem1)
            pltpu.make_async_copy(g_hbm.at[idx_s0], rows0, sem0).wait()
            pltpu.sync_copy(rows0, acc_sh.at[idx_d0], add=True)
            load_i(2 * k + 2, idx_s0, idx_d0)
            pltpu.async_copy(g_hbm.at[idx_s0], rows0, sem0)
            pltpu.make_async_copy(g_hbm.at[idx_s1], rows1, sem1).wait()
            pltpu.sync_copy(rows1, acc_sh.at[idx_d1], add=True)
            return carry

        lax.fori_loop(0, npair, body, 0)
        pltpu.make_async_copy(g_hbm.at[idx_s0], rows0, sem0).wait()
        pltpu.sync_copy(rows0, acc_sh.at[idx_d0], add=True)
        plsc.subcore_barrier()

        def cout(j, carry):
            sl = pl.ds(rbase + j * CH, CH)
            pltpu.sync_copy(acc_sh.at[sl], rows0)
            pltpu.sync_copy(rows0, out_hbm.at[c, sl])
            return carry

        lax.fori_loop(0, nfan, cout, 0)

    return segsum_kernel


def _make_gat_kernel(npad, el_pad, per_tile, rows_per_tile):
    nchunks = per_tile // CH
    nbuf = 3
    nouter = nchunks // nbuf

    @functools.partial(
        pl.kernel,
        mesh=_mesh(),
        out_type=jax.ShapeDtypeStruct((NC * 3 * npad,), jnp.float32),
        scratch_types=[
            pltpu.VMEM((nbuf, CH), jnp.int32),
            pltpu.VMEM((nbuf, CH), jnp.int32),
            pltpu.VMEM((nbuf, CH), jnp.float32),
            pltpu.VMEM((nbuf, CH), jnp.float32),
            pltpu.VMEM((nbuf, CH), jnp.float32),
            pltpu.VMEM((nbuf, CH), jnp.float32),
            pltpu.VMEM((nbuf, CH), jnp.float32),
            pltpu.VMEM((nbuf, CH), jnp.float32),
            pltpu.VMEM((nbuf, CH), jnp.float32),
            pltpu.VMEM((rows_per_tile,), jnp.float32),
            pltpu.VMEM((2, 16), jnp.float32),
            pltpu.VMEM_SHARED((npad,), jnp.float32),
            pltpu.VMEM_SHARED((npad,), jnp.float32),
            pltpu.VMEM_SHARED((npad,), jnp.float32),
        ] + [pltpu.SemaphoreType.DMA] * 27,
    )
    def gat_kernel(hl0_hbm, hl1_hbm, hr0_hbm, hr1_hbm, src_hbm, dst_hbm,
                   att_hbm, out_hbm,
                   idx_s, idx_d, a0_v, a1_v, b0_v, b1_v, o0_v, o1_v, o2_v,
                   zbuf, att_v, acc0, acc1, acc2, *sems):
        c = lax.axis_index("c")
        s = lax.axis_index("s")
        w = s * NC + c

        def zwrite(i, carry):
            zbuf[pl.ds(16 * i, 16)] = jnp.zeros((16,), jnp.float32)
            return carry

        lax.fori_loop(0, rows_per_tile // 16, zwrite, 0)
        osl = pl.ds(s * rows_per_tile, rows_per_tile)
        pltpu.sync_copy(zbuf, acc0.at[osl])
        pltpu.sync_copy(zbuf, acc1.at[osl])
        pltpu.sync_copy(zbuf, acc2.at[osl])
        pltpu.sync_copy(att_hbm, att_v)
        plsc.subcore_barrier()
        att0 = att_v[0, :]
        att1 = att_v[1, :]
        base = w * per_tile

        def start_i(b, chunk):
            off = base + chunk * CH
            pltpu.async_copy(src_hbm.at[pl.ds(off, CH)],
                             idx_s.at[b], sems[9 * b + 7])
            pltpu.async_copy(dst_hbm.at[pl.ds(off, CH)],
                             idx_d.at[b], sems[9 * b + 8])

        def wait_i(b, chunk):
            off = base + chunk * CH
            pltpu.make_async_copy(src_hbm.at[pl.ds(off, CH)],
                                  idx_s.at[b], sems[9 * b + 7]).wait()
            pltpu.make_async_copy(dst_hbm.at[pl.ds(off, CH)],
                                  idx_d.at[b], sems[9 * b + 8]).wait()

        def start_g(b):
            isl = idx_s.at[b]
            idl = idx_d.at[b]
            pltpu.async_copy(hl0_hbm.at[idl], a0_v.at[b], sems[9 * b])
            pltpu.async_copy(hl1_hbm.at[idl], a1_v.at[b], sems[9 * b + 1])
            pltpu.async_copy(hr0_hbm.at[isl], b0_v.at[b], sems[9 * b + 2])
            pltpu.async_copy(hr1_hbm.at[isl], b1_v.at[b], sems[9 * b + 3])

        def wait_g(b):
            isl = idx_s.at[b]
            idl = idx_d.at[b]
            pltpu.make_async_copy(hl0_hbm.at[idl], a0_v.at[b],
                                  sems[9 * b]).wait()
            pltpu.make_async_copy(hl1_hbm.at[idl], a1_v.at[b],
                                  sems[9 * b + 1]).wait()
            pltpu.make_async_copy(hr0_hbm.at[isl], b0_v.at[b],
                                  sems[9 * b + 2]).wait()
            pltpu.make_async_copy(hr1_hbm.at[isl], b1_v.at[b],
                                  sems[9 * b + 3]).wait()

        def compute(b):
            for j in range(CH // 16):
                sl = pl.ds(16 * j, 16)
                a0 = a0_v[b, sl]
                a1 = a1_v[b, sl]
                b0 = b0_v[b, sl]
                b1 = b1_v[b, sl]
                z0 = a0 + b0
                z0 = jnp.where(z0 > 0, z0, 0.2 * z0)
                z1 = a1 + b1
                z1 = jnp.where(z1 > 0, z1, 0.2 * z1)
                p = jnp.exp(att0 * z0 + att1 * z1)
                o0_v[b, sl] = p * b0
                o1_v[b, sl] = p * b1
                o2_v[b, sl] = p

        def start_s(b):
            idl = idx_d.at[b]
            pltpu.async_copy(o0_v.at[b], acc0.at[idl], sems[9 * b + 4],
                             add=True)
            pltpu.async_copy(o1_v.at[b], acc1.at[idl], sems[9 * b + 5],
                             add=True)
            pltpu.async_copy(o2_v.at[b], acc2.at[idl], sems[9 * b + 6],
                             add=True)

        def wait_s(b):
            idl = idx_d.at[b]
            pltpu.make_async_copy(o0_v.at[b], acc0.at[idl],
                                  sems[9 * b + 4]).wait()
            pltpu.make_async_copy(o1_v.at[b], acc1.at[idl],
                                  sems[9 * b + 5]).wait()
            pltpu.make_async_copy(o2_v.at[b], acc2.at[idl],
                                  sems[9 * b + 6]).wait()

        for b in range(nbuf):
            start_i(b, b)
        for b in range(nbuf):
            wait_i(b, b)
            start_g(b)

        def outer(o, carry):
            g0 = o * nbuf
            for b in range(nbuf):
                wait_g(b)
                compute(b)
                start_s(b)
            for b in range(nbuf):
                wait_s(b)
                start_i(b, g0 + nbuf + b)
            for b in range(nbuf):
                wait_i(b, g0 + nbuf + b)
                start_g(b)
            return carry

        lax.fori_loop(0, nouter - 1, outer, 0)
        for b in range(nbuf):
            wait_g(b)
            compute(b)
            start_s(b)
        for b in range(nbuf):
            wait_s(b)
        plsc.subcore_barrier()
        ob = c * 3 * npad + s * rows_per_tile
        pltpu.sync_copy(acc0.at[osl], zbuf)
        pltpu.sync_copy(zbuf, out_hbm.at[pl.ds(ob, rows_per_tile)])
        pltpu.sync_copy(acc1.at[osl], zbuf)
        pltpu.sync_copy(zbuf, out_hbm.at[pl.ds(ob + npad, rows_per_tile)])
        pltpu.sync_copy(acc2.at[osl], zbuf)
        pltpu.sync_copy(zbuf, out_hbm.at[pl.ds(ob + 2 * npad, rows_per_tile)])

    return gat_kernel


def _dinv(dega_ref, degb_ref):
    deg = dega_ref[...] + degb_ref[...]
    return jnp.where(deg > 0, lax.rsqrt(deg), 0.0)


def _mm_scale_body(x_ref, w_ref, dega_ref, degb_ref, o_ref):
    o_ref[...] = _dinv(dega_ref, degb_ref) * jnp.dot(
        x_ref[...], w_ref[...], preferred_element_type=jnp.float32)


def _elu_mm_scale_body(sa_ref, sb_ref, dega_ref, degb_ref, w_ref, b_ref, o_ref):
    dinv = _dinv(dega_ref, degb_ref)
    h = dinv * (sa_ref[...] + sb_ref[...]) + b_ref[...]
    h = jnp.where(h > 0, h, jnp.exp(jnp.minimum(h, 0.0)) - 1.0)
    o_ref[...] = dinv * jnp.dot(h, w_ref[...], preferred_element_type=jnp.float32)


def _elu_mm_body(sa_ref, sb_ref, dega_ref, degb_ref, w_ref, b_ref, o_ref):
    dinv = _dinv(dega_ref, degb_ref)
    h = dinv * (sa_ref[...] + sb_ref[...]) + b_ref[...]
    h = jnp.where(h > 0, h, jnp.exp(jnp.minimum(h, 0.0)) - 1.0)
    o_ref[...] = jnp.dot(h, w_ref[...], preferred_element_type=jnp.float32)


def _softmax_norm_body(ga_ref, gb_ref, b_ref, o_ref):
    num = ga_ref[0:2, :] + gb_ref[0:2, :]
    den = ga_ref[2:3, :] + gb_ref[2:3, :]
    o_ref[...] = num / (den + 1e-16) + b_ref[...]


def _tc_call(body, out_shape, *args):
    return pl.pallas_call(body, out_shape=out_shape)(*args)


def kernel(x, edge_index, W0, b0, W1, b1, Wl, Wr, att, b_out):
    n, d = x.shape
    e = edge_index.shape[1]
    el = e + n
    rows_per_tile = -(-(-(-(n + 1) // NS)) // CH) * CH
    npad = NS * rows_per_tile
    per_tile = -(-el // (NW * CH)) * CH
    if (per_tile // CH) % 2 == 0:
        per_tile += CH
    el_pad = NW * per_tile

    loops = jnp.arange(n, dtype=jnp.int32)
    src = jnp.concatenate([edge_index[0].astype(jnp.int32), loops])
    dst = jnp.concatenate([edge_index[1].astype(jnp.int32), loops])
    pad_e = el_pad - el
    src = jnp.concatenate([src, jnp.full((pad_e,), n, jnp.int32)])
    dst = jnp.concatenate([dst, jnp.full((pad_e,), n, jnp.int32)])

    xpad = jnp.zeros((npad, d), jnp.float32).at[:n].set(x)
    att_rep = jnp.broadcast_to(att.reshape(2, 1), (2, 16)).astype(jnp.float32)
    w_gat = jnp.zeros((d, 8), jnp.float32).at[:, 0:2].set(Wl).at[:, 2:4].set(Wr)

    deg_k = _make_deg_kernel(npad, el_pad, per_tile, rows_per_tile)
    seg_k = _make_segsum_kernel(npad, el_pad, per_tile, rows_per_tile)
    gat_k = _make_gat_kernel(npad, el_pad, per_tile, rows_per_tile)

    dacc = deg_k(dst)
    dega = dacc[:npad].reshape(npad, 1)
    degb = dacc[npad:].reshape(npad, 1)

    g0 = _tc_call(_mm_scale_body,
                  jax.ShapeDtypeStruct((npad, D), jnp.float32),
                  xpad, W0, dega, degb)
    s0 = seg_k(g0, src, dst)
    g1 = _tc_call(_elu_mm_scale_body,
                  jax.ShapeDtypeStruct((npad, D), jnp.float32),
                  s0[0], s0[1], dega, degb, W1, b0)
    s1 = seg_k(g1, src, dst)
    t8 = _tc_call(_elu_mm_body,
                  jax.ShapeDtypeStruct((npad, 8), jnp.float32),
                  s1[0], s1[1], dega, degb, w_gat, b1)
    t4 = jnp.transpose(t8[:, :4])
    gacc = gat_k(t4[0], t4[1], t4[2], t4[3], src, dst, att_rep)
    ga = gacc[:3 * npad].reshape(3, npad)
    gb = gacc[3 * npad:].reshape(3, npad)
    res = _tc_call(_softmax_norm_body,
                   jax.ShapeDtypeStruct((2, npad), jnp.float32),
                   ga, gb, b_out.reshape(2, 1).astype(jnp.float32))
    return jnp.transpose(res)[:n]
